# baseline (speedup 1.0000x reference)
import functools

import jax
import jax.numpy as jnp
from jax import lax
from jax.experimental import pallas as pl
from jax.experimental.pallas import tpu as pltpu
from jax.experimental.pallas import tpu_sc as plsc

NC, NS = 2, 16
NW = NC * NS
L = 128
B, NEG, D = 4096, 200, 64

TAIL_ROWS = B * NEG
TAIL_IR = TAIL_ROWS // L
TAIL_IR_W = TAIL_IR // NW
GJ = 8
GROUPS = TAIL_IR_W // GJ

_mesh = plsc.VectorSubcoreMesh(
    core_axis_name="c", subcore_axis_name="s", num_cores=NC, num_subcores=NS)


@functools.partial(
    pl.kernel,
    out_type=(
        jax.ShapeDtypeStruct((B, D), jnp.float32),
        jax.ShapeDtypeStruct((B, D), jnp.float32),
        jax.ShapeDtypeStruct((TAIL_ROWS, D), jnp.float32),
    ),
    mesh=_mesh,
    scratch_types=[
        pltpu.VMEM((GJ, L), jnp.int32),
        pltpu.VMEM((GJ * L, D), jnp.float32),
        pltpu.SemaphoreType.DMA,
    ],
    compiler_params=pltpu.CompilerParams(use_tc_tiling_on_sc=False),
)
def _gather_kernel(head_idx, rel_idx, tail_idx, ent, rel,
                   head_out, rel_out, tail_out, idx_v, rows_v, sem):
    wid = lax.axis_index("s") * NC + lax.axis_index("c")

    pltpu.sync_copy(head_idx.at[wid], idx_v.at[0])
    pltpu.async_copy(ent.at[idx_v.at[0]], rows_v.at[pl.ds(0, L)], sem).wait()
    pltpu.sync_copy(rows_v.at[pl.ds(0, L)], head_out.at[pl.ds(wid * L, L)])

    pltpu.sync_copy(rel_idx.at[wid], idx_v.at[0])
    pltpu.async_copy(rel.at[idx_v.at[0]], rows_v.at[pl.ds(0, L)], sem).wait()
    pltpu.sync_copy(rows_v.at[pl.ds(0, L)], rel_out.at[pl.ds(wid * L, L)])

    def body(g, carry):
        ir0 = wid * TAIL_IR_W + g * GJ
        pltpu.sync_copy(tail_idx.at[pl.ds(ir0, GJ)], idx_v)
        copies = [
            pltpu.async_copy(ent.at[idx_v.at[j]],
                             rows_v.at[pl.ds(j * L, L)], sem)
            for j in range(GJ)
        ]
        for c in copies:
            c.wait()
        pltpu.sync_copy(rows_v, tail_out.at[pl.ds(ir0 * L, GJ * L)])
        return carry

    lax.fori_loop(0, GROUPS, body, 0)


def kernel(sample, negative_sample, entity_embedding, relation_embedding):
    head_idx = sample[:, 0].astype(jnp.int32).reshape(B // L, L)
    rel_idx = sample[:, 1].astype(jnp.int32).reshape(B // L, L)
    tail_idx = negative_sample.astype(jnp.int32).reshape(TAIL_IR, L)
    head, relation, tail = _gather_kernel(
        head_idx, rel_idx, tail_idx, entity_embedding, relation_embedding)
    return (head[:, None, :], relation[:, None, :],
            tail.reshape(B, NEG, D))


# speedup vs baseline: 1.0099x; 1.0099x over previous
import functools

import jax
import jax.numpy as jnp
from jax import lax
from jax.experimental import pallas as pl
from jax.experimental.pallas import tpu as pltpu
from jax.experimental.pallas import tpu_sc as plsc

NC, NS = 2, 16
NW = NC * NS
L = 128
B, NEG, D = 4096, 200, 64

TAIL_ROWS = B * NEG
TAIL_IR = TAIL_ROWS // L
TAIL_IR_W = TAIL_IR // NW
GJ = 4
GROUPS = TAIL_IR_W // GJ
GROUP_ROWS = GJ * L

_mesh = plsc.VectorSubcoreMesh(
    core_axis_name="c", subcore_axis_name="s", num_cores=NC, num_subcores=NS)


@functools.partial(
    pl.kernel,
    out_type=(
        jax.ShapeDtypeStruct((B, D), jnp.float32),
        jax.ShapeDtypeStruct((B, D), jnp.float32),
        jax.ShapeDtypeStruct((TAIL_ROWS, D), jnp.float32),
    ),
    mesh=_mesh,
    scratch_types=[
        pltpu.VMEM((L,), jnp.int32),
        pltpu.VMEM((TAIL_IR_W, L), jnp.int32),
        pltpu.VMEM((GROUP_ROWS, D), jnp.float32),
        pltpu.VMEM((GROUP_ROWS, D), jnp.float32),
        pltpu.SemaphoreType.DMA,
        pltpu.SemaphoreType.DMA,
        pltpu.SemaphoreType.DMA,
        pltpu.SemaphoreType.DMA,
        pltpu.SemaphoreType.DMA,
    ],
    compiler_params=pltpu.CompilerParams(use_tc_tiling_on_sc=False),
)
def _gather_kernel(head_idx, rel_idx, tail_idx, ent, rel,
                   head_out, rel_out, tail_out,
                   hidx_v, idx_v, rows0, rows1,
                   isem, gsem0, gsem1, wsem0, wsem1):
    wid = lax.axis_index("s") * NC + lax.axis_index("c")
    bufs = (rows0, rows1)
    gsems = (gsem0, gsem1)
    wsems = (wsem0, wsem1)

    idx_cp = pltpu.async_copy(
        tail_idx.at[pl.ds(wid * TAIL_IR_W, TAIL_IR_W)], idx_v, isem)

    pltpu.sync_copy(head_idx.at[wid], hidx_v)
    pltpu.async_copy(ent.at[hidx_v], rows0.at[pl.ds(0, L)], gsem0).wait()
    pltpu.async_copy(rows0.at[pl.ds(0, L)],
                     head_out.at[pl.ds(wid * L, L)], wsem0).wait()

    pltpu.sync_copy(rel_idx.at[wid], hidx_v)
    pltpu.async_copy(rel.at[hidx_v], rows1.at[pl.ds(0, L)], gsem1).wait()
    pltpu.async_copy(rows1.at[pl.ds(0, L)],
                     rel_out.at[pl.ds(wid * L, L)], wsem1).wait()

    idx_cp.wait()

    def out_slice(g):
        return tail_out.at[pl.ds((wid * TAIL_IR_W + g * GJ) * L, GROUP_ROWS)]

    def fire_gathers(g, b):
        for j in range(GJ):
            pltpu.async_copy(ent.at[idx_v.at[g * GJ + j]],
                             bufs[b].at[pl.ds(j * L, L)], gsems[b])

    def drain_gathers(g, b):
        for j in range(GJ):
            pltpu.make_async_copy(ent.at[idx_v.at[g * GJ + j]],
                                  bufs[b].at[pl.ds(j * L, L)], gsems[b]).wait()

    def fire_writeback(g, b):
        pltpu.async_copy(bufs[b], out_slice(g), wsems[b])

    def drain_writeback(g, b):
        pltpu.make_async_copy(bufs[b], out_slice(g), wsems[b]).wait()

    fire_gathers(0, 0)
    fire_gathers(1, 1)
    drain_gathers(0, 0)
    fire_writeback(0, 0)
    drain_gathers(1, 1)
    fire_writeback(1, 1)

    def body(p, carry):
        g0 = 2 + 2 * p
        g1 = g0 + 1
        drain_writeback(g0 - 2, 0)
        fire_gathers(g0, 0)
        drain_writeback(g1 - 2, 1)
        fire_gathers(g1, 1)
        drain_gathers(g0, 0)
        fire_writeback(g0, 0)
        drain_gathers(g1, 1)
        fire_writeback(g1, 1)
        return carry

    lax.fori_loop(0, (GROUPS - 2) // 2, body, 0)

    drain_writeback(GROUPS - 2, 0)
    drain_writeback(GROUPS - 1, 1)


def kernel(sample, negative_sample, entity_embedding, relation_embedding):
    head_idx = sample[:, 0].astype(jnp.int32).reshape(B // L, L)
    rel_idx = sample[:, 1].astype(jnp.int32).reshape(B // L, L)
    tail_idx = negative_sample.astype(jnp.int32).reshape(TAIL_IR, L)
    head, relation, tail = _gather_kernel(
        head_idx, rel_idx, tail_idx, entity_embedding, relation_embedding)
    return (head[:, None, :], relation[:, None, :],
            tail.reshape(B, NEG, D))
